# baseline (device time: 58344 ns/iter reference)
import functools

import jax
import jax.numpy as jnp
from jax import lax
from jax.experimental import pallas as pl
from jax.experimental.pallas import tpu as pltpu

N_DEV = 32
N_TOK = 2048
N_EXP = 128
CAP = 12
E_LOC = N_EXP // N_DEV
SLOTS = E_LOC * CAP
TOK_LOC = N_TOK // N_DEV
D_IN = 512
D_OUT = 1024


def kernel(x, router_W, route_idx, expert_W):
    del router_W
    my = lax.axis_index("i")

    e = route_idx[:, 0]
    onehot = (e[:, None] == jnp.arange(N_EXP)[None, :]).astype(jnp.int32)
    csum = jnp.cumsum(onehot, axis=0)
    rank = jnp.take_along_axis(csum, e[:, None], axis=1)[:, 0] - 1
    kept = (rank < CAP).astype(jnp.int32)
    total = csum[-1]

    sl = jnp.arange(SLOTS, dtype=jnp.int32)
    le = sl // CAP
    r = sl % CAP
    eid = (my * E_LOC + le).astype(jnp.int32)
    valid = (r < total[eid]).astype(jnp.int32)
    match = (e[None, :] == eid[:, None]) & (rank[None, :] == r[:, None])
    tok = jnp.argmax(match, axis=1).astype(jnp.int32)
    dst_dev = (tok // TOK_LOC).astype(jnp.int32)
    dst_row = (tok % TOK_LOC).astype(jnp.int32)
    nrecv = jnp.sum(
        lax.dynamic_slice_in_dim(kept, my * TOK_LOC, TOK_LOC)
    ).astype(jnp.int32).reshape(1)

    def body(x_ref, w_ref, tok_ref, valid_ref, dd_ref, dr_ref, nrecv_ref,
             out_ref, xg_ref, y_ref, send_sems, recv_sem):
        me = lax.axis_index("i")

        out_ref[...] = jnp.zeros_like(out_ref)

        bar = pltpu.get_barrier_semaphore()
        for off in range(1, N_DEV):
            pl.semaphore_signal(
                bar, inc=1,
                device_id=((me + off) % N_DEV,),
                device_id_type=pl.DeviceIdType.MESH,
            )
        pl.semaphore_wait(bar, N_DEV - 1)

        for s in range(SLOTS):
            xg_ref[pl.ds(s, 1), :] = x_ref[pl.ds(tok_ref[s], 1), :]

        for l in range(E_LOC):
            y_ref[l * CAP:(l + 1) * CAP, :] = jnp.dot(
                xg_ref[l * CAP:(l + 1) * CAP, :], w_ref[l],
                preferred_element_type=jnp.float32,
            )

        descs = []
        for s in range(SLOTS):
            descs.append(pltpu.make_async_remote_copy(
                src_ref=y_ref.at[pl.ds(s, 1), :],
                dst_ref=out_ref.at[pl.ds(dr_ref[s], 1), :],
                send_sem=send_sems.at[s],
                recv_sem=recv_sem,
                device_id=(dd_ref[s],),
                device_id_type=pl.DeviceIdType.MESH,
            ))
        for s in range(SLOTS):
            @pl.when(valid_ref[s] != 0)
            def _(s=s):
                descs[s].start()
        for s in range(SLOTS):
            @pl.when(valid_ref[s] != 0)
            def _(s=s):
                descs[s].wait_send()

        recv_d = pltpu.make_async_remote_copy(
            src_ref=y_ref.at[pl.ds(0, 1), :],
            dst_ref=out_ref.at[pl.ds(0, 1), :],
            send_sem=send_sems.at[0],
            recv_sem=recv_sem,
            device_id=(me,),
            device_id_type=pl.DeviceIdType.MESH,
        )

        def rbody(i, carry):
            recv_d.wait_recv()
            return carry

        lax.fori_loop(0, nrecv_ref[0], rbody, 0)

        @functools.partial(pl.run_scoped, sem2=pltpu.SemaphoreType.REGULAR)
        def _(sem2):
            for off in range(1, N_DEV):
                pl.semaphore_signal(
                    sem2, inc=1,
                    device_id=((me + off) % N_DEV,),
                    device_id_type=pl.DeviceIdType.MESH,
                )
            pl.semaphore_wait(sem2, N_DEV - 1)

    return pl.pallas_call(
        body,
        out_shape=jax.ShapeDtypeStruct((TOK_LOC, D_OUT), jnp.float32),
        in_specs=[
            pl.BlockSpec(memory_space=pltpu.VMEM),
            pl.BlockSpec(memory_space=pltpu.VMEM),
            pl.BlockSpec(memory_space=pltpu.SMEM),
            pl.BlockSpec(memory_space=pltpu.SMEM),
            pl.BlockSpec(memory_space=pltpu.SMEM),
            pl.BlockSpec(memory_space=pltpu.SMEM),
            pl.BlockSpec(memory_space=pltpu.SMEM),
        ],
        out_specs=pl.BlockSpec(memory_space=pltpu.VMEM),
        scratch_shapes=[
            pltpu.VMEM((SLOTS, D_IN), jnp.float32),
            pltpu.VMEM((SLOTS, D_OUT), jnp.float32),
            pltpu.SemaphoreType.DMA((SLOTS,)),
            pltpu.SemaphoreType.DMA,
        ],
        compiler_params=pltpu.CompilerParams(collective_id=0),
    )(x, expert_W, tok, valid, dst_dev, dst_row, nrecv)


# device time: 31578 ns/iter; 1.8476x vs baseline; 1.8476x over previous
import functools

import jax
import jax.numpy as jnp
from jax import lax
from jax.experimental import pallas as pl
from jax.experimental.pallas import tpu as pltpu

N_DEV = 32
N_TOK = 2048
N_EXP = 128
CAP = 12
E_LOC = N_EXP // N_DEV
SLOTS = E_LOC * CAP
TOK_LOC = N_TOK // N_DEV
D_IN = 512
D_OUT = 1024


def _lane_cumsum(x):
    c = x
    lane = lax.broadcasted_iota(jnp.int32, x.shape, 1)
    k = 1
    while k < x.shape[1]:
        c = c + jnp.where(lane >= k, jnp.roll(c, k, axis=1), 0)
        k *= 2
    return c


def _prep_body(e_row_ref, ridx_ref, tok_ref, valid_ref, dd_ref, dr_ref,
               nrecv_ref):
    me = lax.axis_index("i")

    s_iota = lax.broadcasted_iota(jnp.int32, (SLOTS, 1), 0)
    eid = me * E_LOC + s_iota // CAP
    r = s_iota % CAP

    e_row = e_row_ref[:, :]
    em = (e_row == eid).astype(jnp.int32)
    cum = _lane_cumsum(em)
    total = cum[:, N_TOK - 1:N_TOK]
    valid = (r < total).astype(jnp.int32)

    lane = lax.broadcasted_iota(jnp.int32, (SLOTS, N_TOK), 1)
    hit = (em == 1) & (cum == r + 1)
    tok = jnp.min(jnp.where(hit, lane, 2 * N_TOK), axis=1, keepdims=True)
    tok = jnp.where(valid == 1, tok, 0)

    tok_ref[:, :] = tok
    valid_ref[:, :] = valid
    dd_ref[:, :] = tok // TOK_LOC
    dr_ref[:, :] = tok % TOK_LOC

    ridx = ridx_ref[:, :]
    eids = lax.broadcasted_iota(jnp.int32, (N_TOK, N_EXP), 1)
    onehot = (ridx == eids).astype(jnp.int32)
    t_iota = lax.broadcasted_iota(jnp.int32, (N_TOK, N_EXP), 0)
    start = me * TOK_LOC
    a = jnp.sum(onehot * (t_iota < start), axis=0, keepdims=True)
    b = jnp.sum(onehot * (t_iota < start + TOK_LOC), axis=0, keepdims=True)
    kept = jnp.maximum(jnp.minimum(b, CAP) - jnp.minimum(a, CAP), 0)
    nrecv_ref[:, :] = jnp.sum(kept, axis=1, keepdims=True)


def _main_body(x_ref, w_ref, tok_ref, valid_ref, dd_ref, dr_ref, nrecv_ref,
               out_ref, xg_ref, y_ref, send_sems, recv_sem):
    me = lax.axis_index("i")

    out_ref[...] = jnp.zeros_like(out_ref)

    bar = pltpu.get_barrier_semaphore()
    for off in range(1, N_DEV):
        pl.semaphore_signal(
            bar, inc=1,
            device_id=((me + off) % N_DEV,),
            device_id_type=pl.DeviceIdType.MESH,
        )
    pl.semaphore_wait(bar, N_DEV - 1)

    for s in range(SLOTS):
        xg_ref[pl.ds(s, 1), :] = x_ref[pl.ds(tok_ref[s, 0], 1), :]

    for l in range(E_LOC):
        y_ref[l * CAP:(l + 1) * CAP, :] = jnp.dot(
            xg_ref[l * CAP:(l + 1) * CAP, :], w_ref[l],
            preferred_element_type=jnp.float32,
        )

    descs = []
    for s in range(SLOTS):
        descs.append(pltpu.make_async_remote_copy(
            src_ref=y_ref.at[pl.ds(s, 1), :],
            dst_ref=out_ref.at[pl.ds(dr_ref[s, 0], 1), :],
            send_sem=send_sems.at[s],
            recv_sem=recv_sem,
            device_id=(dd_ref[s, 0],),
            device_id_type=pl.DeviceIdType.MESH,
        ))
    for s in range(SLOTS):
        @pl.when(valid_ref[s, 0] != 0)
        def _(s=s):
            descs[s].start()
    for s in range(SLOTS):
        @pl.when(valid_ref[s, 0] != 0)
        def _(s=s):
            descs[s].wait_send()

    recv_d = pltpu.make_async_remote_copy(
        src_ref=y_ref.at[pl.ds(0, 1), :],
        dst_ref=out_ref.at[pl.ds(0, 1), :],
        send_sem=send_sems.at[0],
        recv_sem=recv_sem,
        device_id=(me,),
        device_id_type=pl.DeviceIdType.MESH,
    )

    def rbody(i, carry):
        recv_d.wait_recv()
        return carry

    lax.fori_loop(0, nrecv_ref[0, 0], rbody, 0)

    @functools.partial(pl.run_scoped, sem2=pltpu.SemaphoreType.REGULAR)
    def _(sem2):
        for off in range(1, N_DEV):
            pl.semaphore_signal(
                sem2, inc=1,
                device_id=((me + off) % N_DEV,),
                device_id_type=pl.DeviceIdType.MESH,
            )
        pl.semaphore_wait(sem2, N_DEV - 1)


def kernel(x, router_W, route_idx, expert_W):
    del router_W

    e_row = route_idx.reshape(1, N_TOK)

    i32 = jnp.int32
    tok, valid, dd, dr, nrecv = pl.pallas_call(
        _prep_body,
        out_shape=[
            jax.ShapeDtypeStruct((SLOTS, 1), i32),
            jax.ShapeDtypeStruct((SLOTS, 1), i32),
            jax.ShapeDtypeStruct((SLOTS, 1), i32),
            jax.ShapeDtypeStruct((SLOTS, 1), i32),
            jax.ShapeDtypeStruct((1, 1), i32),
        ],
        in_specs=[
            pl.BlockSpec(memory_space=pltpu.VMEM),
            pl.BlockSpec(memory_space=pltpu.VMEM),
        ],
        out_specs=[pl.BlockSpec(memory_space=pltpu.VMEM)] * 5,
    )(e_row, route_idx)

    return pl.pallas_call(
        _main_body,
        out_shape=jax.ShapeDtypeStruct((TOK_LOC, D_OUT), jnp.float32),
        in_specs=[
            pl.BlockSpec(memory_space=pltpu.VMEM),
            pl.BlockSpec(memory_space=pltpu.VMEM),
            pl.BlockSpec(memory_space=pltpu.SMEM),
            pl.BlockSpec(memory_space=pltpu.SMEM),
            pl.BlockSpec(memory_space=pltpu.SMEM),
            pl.BlockSpec(memory_space=pltpu.SMEM),
            pl.BlockSpec(memory_space=pltpu.SMEM),
        ],
        out_specs=pl.BlockSpec(memory_space=pltpu.VMEM),
        scratch_shapes=[
            pltpu.VMEM((SLOTS, D_IN), jnp.float32),
            pltpu.VMEM((SLOTS, D_OUT), jnp.float32),
            pltpu.SemaphoreType.DMA((SLOTS,)),
            pltpu.SemaphoreType.DMA,
        ],
        compiler_params=pltpu.CompilerParams(collective_id=0),
    )(x, expert_W, tok, valid, dd, dr, nrecv)


# device time: 22335 ns/iter; 2.6122x vs baseline; 1.4138x over previous
import jax
import jax.numpy as jnp
from jax import lax
from jax.experimental import pallas as pl
from jax.experimental.pallas import tpu as pltpu

N_DEV = 32
N_TOK = 2048
N_EXP = 128
CAP = 12
E_LOC = N_EXP // N_DEV
SLOTS = E_LOC * CAP
TOK_LOC = N_TOK // N_DEV
D_IN = 512
D_OUT = 1024


def _lane_cumsum(x):
    c = x
    lane = lax.broadcasted_iota(jnp.int32, x.shape, 1)
    k = 1
    while k < x.shape[1]:
        c = c + jnp.where(lane >= k, jnp.roll(c, k, axis=1), 0)
        k *= 2
    return c


def _body(x_ref, w_ref, e_row_ref, ridx_ref, out_ref,
          xg_ref, y_ref, meta_vmem, meta_smem, send_sems, recv_sem,
          meta_sem):
    me = lax.axis_index("i")

    out_ref[...] = jnp.zeros_like(out_ref)

    bar = pltpu.get_barrier_semaphore()
    for off in range(1, N_DEV):
        pl.semaphore_signal(
            bar, inc=1,
            device_id=((me + off) % N_DEV,),
            device_id_type=pl.DeviceIdType.MESH,
        )

    s_iota = lax.broadcasted_iota(jnp.int32, (SLOTS, 1), 0)
    eid = me * E_LOC + s_iota // CAP
    r = s_iota % CAP

    e_row = e_row_ref[:, :]
    em = (e_row == eid).astype(jnp.int32)
    cum = _lane_cumsum(em)
    total = cum[:, N_TOK - 1:N_TOK]
    valid = r < total

    lane = lax.broadcasted_iota(jnp.int32, (SLOTS, N_TOK), 1)
    hit = (em == 1) & (cum == r + 1)
    tok = jnp.min(jnp.where(hit, lane, 2 * N_TOK), axis=1, keepdims=True)
    tok_enc = jnp.where(valid, tok, -1)

    ridx = ridx_ref[:, :]
    eids = lax.broadcasted_iota(jnp.int32, (N_TOK, N_EXP), 1)
    onehot = (ridx == eids).astype(jnp.int32)
    t_iota = lax.broadcasted_iota(jnp.int32, (N_TOK, N_EXP), 0)
    start = me * TOK_LOC
    a = jnp.sum(onehot * (t_iota < start), axis=0, keepdims=True)
    b = jnp.sum(onehot * (t_iota < start + TOK_LOC), axis=0, keepdims=True)
    kept = jnp.maximum(jnp.minimum(b, CAP) - jnp.minimum(a, CAP), 0)
    nrecv = jnp.sum(kept, axis=1, keepdims=True)

    meta_vmem[0:SLOTS, :] = tok_enc
    meta_vmem[SLOTS:SLOTS + 1, :] = nrecv
    cp = pltpu.make_async_copy(meta_vmem, meta_smem, meta_sem)
    cp.start()
    cp.wait()

    for s in range(SLOTS):
        t = jnp.maximum(meta_smem[s, 0], 0)
        xg_ref[pl.ds(s, 1), :] = x_ref[pl.ds(t, 1), :]

    for l in range(E_LOC):
        y_ref[l * CAP:(l + 1) * CAP, :] = jnp.dot(
            xg_ref[l * CAP:(l + 1) * CAP, :], w_ref[l],
            preferred_element_type=jnp.float32,
        )

    pl.semaphore_wait(bar, N_DEV - 1)

    descs = []
    for s in range(SLOTS):
        t = meta_smem[s, 0]
        descs.append((t, pltpu.make_async_remote_copy(
            src_ref=y_ref.at[pl.ds(s, 1), :],
            dst_ref=out_ref.at[pl.ds(lax.rem(t, TOK_LOC), 1), :],
            send_sem=send_sems.at[s],
            recv_sem=recv_sem,
            device_id=(lax.div(t, TOK_LOC),),
            device_id_type=pl.DeviceIdType.MESH,
        )))
    for s in range(SLOTS):
        @pl.when(descs[s][0] >= 0)
        def _(s=s):
            descs[s][1].start()
    for s in range(SLOTS):
        @pl.when(descs[s][0] >= 0)
        def _(s=s):
            descs[s][1].wait_send()

    recv_d = pltpu.make_async_remote_copy(
        src_ref=y_ref.at[pl.ds(0, 1), :],
        dst_ref=out_ref.at[pl.ds(0, 1), :],
        send_sem=send_sems.at[0],
        recv_sem=recv_sem,
        device_id=(me,),
        device_id_type=pl.DeviceIdType.MESH,
    )

    def rbody(i, carry):
        recv_d.wait_recv()
        return carry

    lax.fori_loop(0, meta_smem[SLOTS, 0], rbody, 0)


def kernel(x, router_W, route_idx, expert_W):
    del router_W

    e_row = route_idx.reshape(1, N_TOK)

    return pl.pallas_call(
        _body,
        out_shape=jax.ShapeDtypeStruct((TOK_LOC, D_OUT), jnp.float32),
        in_specs=[
            pl.BlockSpec(memory_space=pltpu.VMEM),
            pl.BlockSpec(memory_space=pltpu.VMEM),
            pl.BlockSpec(memory_space=pltpu.VMEM),
            pl.BlockSpec(memory_space=pltpu.VMEM),
        ],
        out_specs=pl.BlockSpec(memory_space=pltpu.VMEM),
        scratch_shapes=[
            pltpu.VMEM((SLOTS, D_IN), jnp.float32),
            pltpu.VMEM((SLOTS, D_OUT), jnp.float32),
            pltpu.VMEM((SLOTS + 1, 1), jnp.int32),
            pltpu.SMEM((SLOTS + 1, 1), jnp.int32),
            pltpu.SemaphoreType.DMA((SLOTS,)),
            pltpu.SemaphoreType.DMA,
            pltpu.SemaphoreType.DMA,
        ],
        compiler_params=pltpu.CompilerParams(collective_id=0),
    )(x, expert_W, e_row, route_idx)


# device time: 20875 ns/iter; 2.7949x vs baseline; 1.0699x over previous
import jax
import jax.numpy as jnp
from jax import lax
from jax.experimental import pallas as pl
from jax.experimental.pallas import tpu as pltpu

N_DEV = 32
N_TOK = 2048
N_EXP = 128
CAP = 12
E_LOC = N_EXP // N_DEV
SLOTS = E_LOC * CAP
TOK_LOC = N_TOK // N_DEV
D_IN = 512
D_OUT = 1024


def _lane_cumsum(x):
    c = x
    lane = lax.broadcasted_iota(jnp.int32, x.shape, 1)
    k = 1
    while k < x.shape[1]:
        c = c + jnp.where(lane >= k, jnp.roll(c, k, axis=1), 0)
        k *= 2
    return c


def _body(x_ref, w_ref, e_row_ref, out_ref,
          xg_ref, y_ref, meta_vmem, meta_smem, send_sems, recv_sem,
          meta_sem):
    me = lax.axis_index("i")

    out_ref[...] = jnp.zeros_like(out_ref)

    bar = pltpu.get_barrier_semaphore()
    for off in range(1, N_DEV):
        pl.semaphore_signal(
            bar, inc=1,
            device_id=((me + off) % N_DEV,),
            device_id_type=pl.DeviceIdType.MESH,
        )

    s_iota = lax.broadcasted_iota(jnp.int32, (SLOTS, 1), 0)
    eid = me * E_LOC + s_iota // CAP
    r = s_iota % CAP

    e_row = e_row_ref[:, :]
    em = (e_row == eid).astype(jnp.int32)
    cum = _lane_cumsum(em)
    total = cum[:, N_TOK - 1:N_TOK]
    valid = r < total

    lane = lax.broadcasted_iota(jnp.int32, (SLOTS, N_TOK), 1)
    hit = (em == 1) & (cum == r + 1)
    tok = jnp.min(jnp.where(hit, lane, 2 * N_TOK), axis=1, keepdims=True)
    tok_enc = jnp.where(valid, tok, -1)

    e_col = lax.broadcasted_iota(jnp.int32, (N_EXP, 1), 0)
    ee = (e_row == e_col)
    lane2 = lax.broadcasted_iota(jnp.int32, (N_EXP, N_TOK), 1)
    start = me * TOK_LOC
    a = jnp.sum((ee & (lane2 < start)).astype(jnp.int32), axis=1,
                keepdims=True)
    b = jnp.sum((ee & (lane2 < start + TOK_LOC)).astype(jnp.int32), axis=1,
                keepdims=True)
    kept = jnp.maximum(jnp.minimum(b, CAP) - jnp.minimum(a, CAP), 0)
    nrecv = jnp.sum(kept, axis=0, keepdims=True)

    meta_vmem[0:SLOTS, :] = tok_enc
    meta_vmem[SLOTS:SLOTS + 1, :] = nrecv
    cp = pltpu.make_async_copy(meta_vmem, meta_smem, meta_sem)
    cp.start()
    cp.wait()

    for s in range(SLOTS):
        t = jnp.maximum(meta_smem[s, 0], 0)
        xg_ref[pl.ds(s, 1), :] = x_ref[pl.ds(t, 1), :]

    for l in range(E_LOC):
        y_ref[l * CAP:(l + 1) * CAP, :] = jnp.dot(
            xg_ref[l * CAP:(l + 1) * CAP, :], w_ref[l],
            preferred_element_type=jnp.float32,
        )

    pl.semaphore_wait(bar, N_DEV - 1)

    descs = []
    for s in range(SLOTS):
        t = meta_smem[s, 0]
        descs.append((t, pltpu.make_async_remote_copy(
            src_ref=y_ref.at[pl.ds(s, 1), :],
            dst_ref=out_ref.at[pl.ds(lax.rem(t, TOK_LOC), 1), :],
            send_sem=send_sems.at[s],
            recv_sem=recv_sem,
            device_id=(lax.div(t, TOK_LOC),),
            device_id_type=pl.DeviceIdType.MESH,
        )))
    for s in range(SLOTS):
        @pl.when(descs[s][0] >= 0)
        def _(s=s):
            descs[s][1].start()
    for s in range(SLOTS):
        @pl.when(descs[s][0] >= 0)
        def _(s=s):
            descs[s][1].wait_send()

    recv_d = pltpu.make_async_remote_copy(
        src_ref=y_ref.at[pl.ds(0, 1), :],
        dst_ref=out_ref.at[pl.ds(0, 1), :],
        send_sem=send_sems.at[0],
        recv_sem=recv_sem,
        device_id=(me,),
        device_id_type=pl.DeviceIdType.MESH,
    )

    def rbody(i, carry):
        recv_d.wait_recv()
        return carry

    lax.fori_loop(0, meta_smem[SLOTS, 0], rbody, 0)


def kernel(x, router_W, route_idx, expert_W):
    del router_W

    e_row = route_idx.reshape(1, N_TOK)

    return pl.pallas_call(
        _body,
        out_shape=jax.ShapeDtypeStruct((TOK_LOC, D_OUT), jnp.float32),
        in_specs=[
            pl.BlockSpec(memory_space=pltpu.VMEM),
            pl.BlockSpec(memory_space=pltpu.VMEM),
            pl.BlockSpec(memory_space=pltpu.VMEM),
        ],
        out_specs=pl.BlockSpec(memory_space=pltpu.VMEM),
        scratch_shapes=[
            pltpu.VMEM((SLOTS, D_IN), jnp.float32),
            pltpu.VMEM((SLOTS, D_OUT), jnp.float32),
            pltpu.VMEM((SLOTS + 1, 1), jnp.int32),
            pltpu.SMEM((SLOTS + 1, 1), jnp.int32),
            pltpu.SemaphoreType.DMA((SLOTS,)),
            pltpu.SemaphoreType.DMA,
            pltpu.SemaphoreType.DMA,
        ],
        compiler_params=pltpu.CompilerParams(collective_id=0),
    )(x, expert_W, e_row)
